# Initial kernel scaffold; baseline (speedup 1.0000x reference)
import jax
import jax.numpy as jnp
from jax.experimental import pallas as pl


def kernel(tile_feats, piece_feats, global_feats, tile_edge_index, piece_to_tile, tile_to_piece, B, T, P, params):
    raise NotImplementedError("write your pallas kernel here")


# speedup vs baseline: 1.5078x; 1.5078x over previous
import functools

import jax
import jax.numpy as jnp
from jax.experimental import pallas as pl

H = 64


def _lin_body(x_ref, w_ref, b_ref, o_ref, *, relu):
    y = jnp.dot(x_ref[...], w_ref[...], preferred_element_type=jnp.float32)
    y = y + b_ref[...]
    o_ref[...] = jnp.maximum(y, 0.0) if relu else y


def _lin(x, w, b, relu, block=512):
    R, K = x.shape
    N = w.shape[1]
    return pl.pallas_call(
        functools.partial(_lin_body, relu=relu),
        grid=(R // block,),
        in_specs=[
            pl.BlockSpec((block, K), lambda i: (i, 0)),
            pl.BlockSpec((K, N), lambda i: (0, 0)),
            pl.BlockSpec((1, N), lambda i: (0, 0)),
        ],
        out_specs=pl.BlockSpec((block, N), lambda i: (i, 0)),
        out_shape=jax.ShapeDtypeStruct((R, N), jnp.float32),
    )(x, w, b)


def _upd2_body(x1_ref, agg_ref, cnt_ref, w_ref, b_ref, bm_ref, o_ref):
    c = cnt_ref[...]
    recip = 1.0 / jnp.maximum(c, 1.0)
    m = jnp.minimum(c, 1.0)
    x2 = agg_ref[...] * recip + m * bm_ref[...]
    x = jnp.concatenate([x1_ref[...], x2], axis=1)
    y = jnp.dot(x, w_ref[...], preferred_element_type=jnp.float32) + b_ref[...]
    o_ref[...] = jnp.maximum(y, 0.0)


def _upd2(x1, agg, cnt, w, b, bm, block=512):
    R = x1.shape[0]
    return pl.pallas_call(
        _upd2_body,
        grid=(R // block,),
        in_specs=[
            pl.BlockSpec((block, H), lambda i: (i, 0)),
            pl.BlockSpec((block, H), lambda i: (i, 0)),
            pl.BlockSpec((block, 1), lambda i: (i, 0)),
            pl.BlockSpec((2 * H, H), lambda i: (0, 0)),
            pl.BlockSpec((1, H), lambda i: (0, 0)),
            pl.BlockSpec((1, H), lambda i: (0, 0)),
        ],
        out_specs=pl.BlockSpec((block, H), lambda i: (i, 0)),
        out_shape=jax.ShapeDtypeStruct((R, H), jnp.float32),
    )(x1, agg, cnt, w, b, bm)


def _upd_body(x1_ref, x2_ref, w_ref, b_ref, o_ref):
    x = jnp.concatenate([x1_ref[...], x2_ref[...]], axis=1)
    y = jnp.dot(x, w_ref[...], preferred_element_type=jnp.float32) + b_ref[...]
    o_ref[...] = jnp.maximum(y, 0.0)


def _upd(x1, x2, w, b, block=512):
    R = x1.shape[0]
    return pl.pallas_call(
        _upd_body,
        grid=(R // block,),
        in_specs=[
            pl.BlockSpec((block, H), lambda i: (i, 0)),
            pl.BlockSpec((block, H), lambda i: (i, 0)),
            pl.BlockSpec((2 * H, H), lambda i: (0, 0)),
            pl.BlockSpec((1, H), lambda i: (0, 0)),
        ],
        out_specs=pl.BlockSpec((block, H), lambda i: (i, 0)),
        out_shape=jax.ShapeDtypeStruct((R, H), jnp.float32),
    )(x1, x2, w, b)


def _tt_body(t_ref, mn_ref, wm_ref, be_ref, o_ref, *, nb, T):
    tb = t_ref[...]
    mn = mn_ref[...]
    wm = wm_ref[...]
    outs = []
    for j in range(nb):
        a = jnp.dot(mn, tb[j], preferred_element_type=jnp.float32)
        outs.append(jnp.dot(a, wm, preferred_element_type=jnp.float32)[None])
    o_ref[...] = jnp.concatenate(outs, axis=0) + be_ref[...]


def _tt(tile3, mn, wm, beff, nb=8):
    B, T, _ = tile3.shape
    return pl.pallas_call(
        functools.partial(_tt_body, nb=nb, T=T),
        grid=(B // nb,),
        in_specs=[
            pl.BlockSpec((nb, T, H), lambda i: (i, 0, 0)),
            pl.BlockSpec((T, T), lambda i: (0, 0)),
            pl.BlockSpec((H, H), lambda i: (0, 0)),
            pl.BlockSpec((1, T, H), lambda i: (0, 0, 0)),
        ],
        out_specs=pl.BlockSpec((nb, T, H), lambda i: (i, 0, 0)),
        out_shape=jax.ShapeDtypeStruct((B, T, H), jnp.float32),
    )(tile3, mn, wm, beff)


def _global_body(t_ref, gh_ref, wg_ref, bg_ref, wu_ref, bu_ref, wm_ref,
                 bm_ref, gh_o, gm_o):
    tmean = jnp.mean(t_ref[...], axis=1)
    ga = jnp.dot(tmean, wg_ref[...], preferred_element_type=jnp.float32) + bg_ref[...]
    x = jnp.concatenate([gh_ref[...], ga], axis=1)
    ghn = jnp.maximum(
        jnp.dot(x, wu_ref[...], preferred_element_type=jnp.float32) + bu_ref[...], 0.0)
    gh_o[...] = ghn
    gm_o[...] = jnp.dot(ghn, wm_ref[...], preferred_element_type=jnp.float32) + bm_ref[...]


def _global(tile3, gh, wg, bg, wu, bu, wm, bm, nb=256):
    B, T, _ = tile3.shape
    nb = min(nb, B)
    return pl.pallas_call(
        _global_body,
        grid=(B // nb,),
        in_specs=[
            pl.BlockSpec((nb, T, H), lambda i: (i, 0, 0)),
            pl.BlockSpec((nb, H), lambda i: (i, 0)),
            pl.BlockSpec((H, H), lambda i: (0, 0)),
            pl.BlockSpec((1, H), lambda i: (0, 0)),
            pl.BlockSpec((2 * H, H), lambda i: (0, 0)),
            pl.BlockSpec((1, H), lambda i: (0, 0)),
            pl.BlockSpec((H, H), lambda i: (0, 0)),
            pl.BlockSpec((1, H), lambda i: (0, 0)),
        ],
        out_specs=[
            pl.BlockSpec((nb, H), lambda i: (i, 0)),
            pl.BlockSpec((nb, H), lambda i: (i, 0)),
        ],
        out_shape=[
            jax.ShapeDtypeStruct((B, H), jnp.float32),
            jax.ShapeDtypeStruct((B, H), jnp.float32),
        ],
    )(tile3, gh, wg, bg, wu, bu, wm, bm)


def _updg_body(t_ref, g_ref, w_ref, b_ref, o_ref, *, nb, T):
    tb = t_ref[...]
    g3 = jnp.broadcast_to(g_ref[...][:, None, :], (nb, T, H))
    x = jnp.concatenate([tb, g3], axis=2).reshape(nb * T, 2 * H)
    y = jnp.dot(x, w_ref[...], preferred_element_type=jnp.float32) + b_ref[...]
    o_ref[...] = jnp.maximum(y, 0.0).reshape(nb, T, H)


def _updg(tile3, gmsg, w, b, nb=8):
    B, T, _ = tile3.shape
    return pl.pallas_call(
        functools.partial(_updg_body, nb=nb, T=T),
        grid=(B // nb,),
        in_specs=[
            pl.BlockSpec((nb, T, H), lambda i: (i, 0, 0)),
            pl.BlockSpec((nb, H), lambda i: (i, 0)),
            pl.BlockSpec((2 * H, H), lambda i: (0, 0)),
            pl.BlockSpec((1, H), lambda i: (0, 0)),
        ],
        out_specs=pl.BlockSpec((nb, T, H), lambda i: (i, 0, 0)),
        out_shape=jax.ShapeDtypeStruct((B, T, H), jnp.float32),
    )(tile3, gmsg, w, b)


def _readout_body(t_ref, p_ref, pi_ref, gh_ref, gf_ref, w0_ref, b0_ref,
                  w1_ref, b1_ref, w2_ref, b2_ref, o_ref):
    tp = jnp.mean(t_ref[...], axis=1)
    pp = (jnp.mean(p_ref[...], axis=1) + jnp.mean(pi_ref[...], axis=1)) * 0.5
    comb = jnp.concatenate([tp, pp, gh_ref[...], gf_ref[...]], axis=1)
    h = jnp.maximum(
        jnp.dot(comb, w0_ref[...], preferred_element_type=jnp.float32) + b0_ref[...], 0.0)
    h = jnp.maximum(
        jnp.dot(h, w1_ref[...], preferred_element_type=jnp.float32) + b1_ref[...], 0.0)
    o_ref[...] = jnp.dot(h, w2_ref[...], preferred_element_type=jnp.float32) + b2_ref[...]


def _readout(tile3, piece3, piece_init3, gh, gf, r0, r1, r2, nb=256):
    B, T, _ = tile3.shape
    nb = min(nb, B)
    P = piece3.shape[1]
    GF = gf.shape[1]
    D0 = 3 * H + GF
    return pl.pallas_call(
        _readout_body,
        grid=(B // nb,),
        in_specs=[
            pl.BlockSpec((nb, T, H), lambda i: (i, 0, 0)),
            pl.BlockSpec((nb, P, H), lambda i: (i, 0, 0)),
            pl.BlockSpec((nb, P, H), lambda i: (i, 0, 0)),
            pl.BlockSpec((nb, H), lambda i: (i, 0)),
            pl.BlockSpec((nb, GF), lambda i: (i, 0)),
            pl.BlockSpec((D0, H), lambda i: (0, 0)),
            pl.BlockSpec((1, H), lambda i: (0, 0)),
            pl.BlockSpec((H, 32), lambda i: (0, 0)),
            pl.BlockSpec((1, 32), lambda i: (0, 0)),
            pl.BlockSpec((32, 1), lambda i: (0, 0)),
            pl.BlockSpec((1, 1), lambda i: (0, 0)),
        ],
        out_specs=pl.BlockSpec((nb, 1), lambda i: (i, 0)),
        out_shape=jax.ShapeDtypeStruct((B, 1), jnp.float32),
    )(tile3, piece3, piece_init3, gh, gf, r0["w"], r0["b"][None],
      r1["w"], r1["b"][None], r2["w"], r2["b"][None])


def _segsum(proj, src, dst, n):
    return jax.ops.segment_sum(proj[src], dst, num_segments=n)


def kernel(tile_feats, piece_feats, global_feats, tile_edge_index,
           piece_to_tile, tile_to_piece, B, T, P, params):
    del B, T, P
    Bs, Ts, TF = tile_feats.shape
    Ps = piece_feats.shape[1]
    BT = Bs * Ts
    BP = Bs * Ps
    E = piece_to_tile.shape[1]

    t2p_src, t2p_dst = tile_to_piece[0], tile_to_piece[1]
    p2t_src, p2t_dst = piece_to_tile[0], piece_to_tile[1]

    ones = jnp.ones((E,), jnp.float32)
    cnt_p = jax.ops.segment_sum(ones, t2p_dst, num_segments=BP)[:, None]
    cnt_t = jax.ops.segment_sum(ones, p2t_dst, num_segments=BT)[:, None]

    src_tt, dst_tt = tile_edge_index[0], tile_edge_index[1]
    ar = jnp.arange(Ts, dtype=jnp.int32)
    ohs = (src_tt[:, None] == ar[None, :]).astype(jnp.float32)
    ohd = (dst_tt[:, None] == ar[None, :]).astype(jnp.float32)
    M = ohd.T @ ohs
    cnt_tt = M.sum(axis=1)
    mn = M / jnp.maximum(cnt_tt, 1.0)[:, None]
    bscale_tt = jnp.minimum(cnt_tt, 1.0)

    te, pe = params["tile_embed"], params["piece_embed"]
    tile_flat = _lin(tile_feats.reshape(BT, TF), te["w"], te["b"][None], relu=True)
    piece_flat = _lin(piece_feats.reshape(BP, -1), pe["w"], pe["b"][None], relu=True)
    piece_init = piece_flat
    gh = jnp.broadcast_to(params["global_embed"], (Bs, H))

    for p in params["mp"]:
        w_tp, b_tp = p["tile_to_piece_msg"]["w"], p["tile_to_piece_msg"]["b"]
        proj = _lin(tile_flat[:BP], w_tp, jnp.zeros((1, H), jnp.float32), relu=False)
        agg_p = _segsum(proj, t2p_src, t2p_dst, BP)
        pu = p["piece_update"]
        piece_flat = _upd2(piece_flat, agg_p, cnt_p, pu["w"], pu["b"][None], b_tp[None])

        w_pt, b_pt = p["piece_to_tile_msg"]["w"], p["piece_to_tile_msg"]["b"]
        proj2 = _lin(piece_flat, w_pt, jnp.zeros((1, H), jnp.float32), relu=False)
        agg_t = _segsum(proj2, p2t_src, p2t_dst, BT)
        tu = p["tile_update_pieces"]
        tile_flat = _upd2(tile_flat, agg_t, cnt_t, tu["w"], tu["b"][None], b_pt[None])

        tile3 = tile_flat.reshape(Bs, Ts, H)
        wm_tt, bm_tt = p["tile_to_tile_msg"]["w"], p["tile_to_tile_msg"]["b"]
        beff = (bscale_tt[:, None] * bm_tt[None, :])[None]
        x2tt = _tt(tile3, mn, wm_tt, beff)
        tt_u = p["tile_update_tiles"]
        tile_flat = _upd(tile_flat, x2tt.reshape(BT, H), tt_u["w"], tt_u["b"][None])
        tile3 = tile_flat.reshape(Bs, Ts, H)

        gmsg_p = p["tile_to_global_msg"]
        gu = p["global_update"]
        g2t = p["global_to_tile_msg"]
        gh, gmsg = _global(tile3, gh, gmsg_p["w"], gmsg_p["b"][None],
                           gu["w"], gu["b"][None], g2t["w"], g2t["b"][None])
        tg_u = p["tile_update_global"]
        tile3 = _updg(tile3, gmsg, tg_u["w"], tg_u["b"][None])
        tile_flat = tile3.reshape(BT, H)

    r0, r1, r2 = params["readout"]
    value = _readout(tile_flat.reshape(Bs, Ts, H), piece_flat.reshape(Bs, Ps, H),
                     piece_init.reshape(Bs, Ps, H), gh, global_feats, r0, r1, r2)
    return value[:, 0]
